# Initial kernel scaffold; baseline (speedup 1.0000x reference)
import jax
import jax.numpy as jnp
from jax.experimental import pallas as pl


def kernel(x, edge_index, batch, d0_Wl1, d0_Wr1, d0_b1, d0_Wl2, d0_Wr2, d0_b2, d1_Wl1, d1_Wr1, d1_b1, d1_Wl2, d1_Wr2, d1_b2, p0, p1, u0_Wl1, u0_Wr1, u0_b1, u0_Wl2, u0_Wr2, u0_b2):
    raise NotImplementedError("write your pallas kernel here")


# speedup vs baseline: 7.2999x; 7.2999x over previous
import functools
import numpy as np
import jax
import jax.numpy as jnp
from jax import lax
from jax.experimental import pallas as pl
from jax.experimental.pallas import tpu as pltpu
from jax.experimental.pallas import tpu_sc as plsc

N0 = 10000
E = 320000
EP = 327680
K1 = 2500
K2 = 625
NP0 = 10240
NP1 = 2560
NW = 32
WE = 128
NWIN = EP // 16 // WE
WP = 80
TR = 256

_MESH = plsc.VectorSubcoreMesh(core_axis_name="c", subcore_axis_name="s")


@functools.lru_cache(maxsize=None)
def _make_agg(n_pad: int):
    half = n_pad // 2
    n_acc = half + TR
    rps = n_acc // 16

    @functools.partial(
        pl.kernel,
        out_type=jax.ShapeDtypeStruct((2, 16, rps, 128), jnp.float32),
        mesh=_MESH,
        scratch_types=(
            pltpu.VMEM((NWIN, WE), jnp.int32),
            pltpu.VMEM((NWIN, WE), jnp.int32),
            pltpu.VMEM((1, WE), jnp.int32),
            pltpu.VMEM((1, WE), jnp.int32),
            pltpu.VMEM((WE, 128), jnp.float32),
            pltpu.VMEM_SHARED((n_acc, 128), jnp.float32),
        ),
    )
    def agg(x_hbm, src_hbm, dst_hbm, z2_hbm, part_hbm,
            src_v, dst_v, dloc_v, srcw_v, rows0, acc_sh):
        c = lax.axis_index("c")
        s = lax.axis_index("s")
        base = c * half
        r0 = s * rps
        pltpu.sync_copy(z2_hbm.at[s], acc_sh.at[pl.ds(r0, rps)])
        pltpu.sync_copy(src_hbm.at[s], src_v)
        pltpu.sync_copy(dst_hbm.at[s], dst_v)
        plsc.subcore_barrier()
        iota = lax.iota(jnp.int32, 16)

        def body(w, carry):
            for l in range(WE // 16):
                srcw_v[0, pl.ds(l * 16, 16)] = src_v[w, pl.ds(l * 16, 16)]
                d16 = dst_v[w, pl.ds(l * 16, 16)] - base
                ok = (d16 >= 0) & (d16 < half)
                trash = half + ((l * 16 + iota) & (TR - 1))
                dloc_v[0, pl.ds(l * 16, 16)] = jnp.where(ok, d16, trash)
            pltpu.sync_copy(x_hbm.at[srcw_v.at[0]], rows0)
            pltpu.sync_copy(rows0, acc_sh.at[dloc_v.at[0]], add=True)
            return carry

        lax.fori_loop(0, NWIN, body, 0)
        plsc.subcore_barrier()
        pltpu.sync_copy(acc_sh.at[pl.ds(r0, rps)], part_hbm.at[c, s])

    return agg


@functools.lru_cache(maxsize=None)
def _make_deg(n_pad: int):
    half = n_pad // 2
    n_acc = half + TR
    rps = n_acc // 16

    @functools.partial(
        pl.kernel,
        out_type=jax.ShapeDtypeStruct((2, 16, rps, 128), jnp.float32),
        mesh=_MESH,
        scratch_types=(
            pltpu.VMEM((NWIN, WE), jnp.int32),
            pltpu.VMEM((1, WE), jnp.int32),
            pltpu.VMEM((WE, 128), jnp.float32),
            pltpu.VMEM_SHARED((n_acc, 128), jnp.float32),
        ),
    )
    def deg(dst_hbm, z2_hbm, ones_hbm, deg_hbm, dst_v, dloc_v, ones_v, acc_sh):
        c = lax.axis_index("c")
        s = lax.axis_index("s")
        base = c * half
        r0 = s * rps
        pltpu.sync_copy(z2_hbm.at[s], acc_sh.at[pl.ds(r0, rps)])
        pltpu.sync_copy(ones_hbm, ones_v)
        pltpu.sync_copy(dst_hbm.at[s], dst_v)
        plsc.subcore_barrier()
        iota = lax.iota(jnp.int32, 16)

        def body(w, carry):
            for l in range(WE // 16):
                d16 = dst_v[w, pl.ds(l * 16, 16)] - base
                ok = (d16 >= 0) & (d16 < half)
                trash = half + ((l * 16 + iota) & (TR - 1))
                dloc_v[0, pl.ds(l * 16, 16)] = jnp.where(ok, d16, trash)
            pltpu.sync_copy(ones_v, acc_sh.at[dloc_v.at[0]], add=True)
            return carry

        lax.fori_loop(0, NWIN, body, 0)
        plsc.subcore_barrier()
        pltpu.sync_copy(acc_sh.at[pl.ds(r0, rps)], deg_hbm.at[c, s])

    return deg


@functools.lru_cache(maxsize=None)
def _make_pool(n_pad: int, k: int):
    nwp = n_pad // NW // WP

    @functools.partial(
        pl.kernel,
        out_type=jax.ShapeDtypeStruct((k + 32, 128), jnp.float32),
        mesh=_MESH,
        scratch_types=(
            pltpu.VMEM((nwp, WP), jnp.int32),
            pltpu.VMEM((WP, 128), jnp.float32),
            pltpu.VMEM((1, WP), jnp.int32),
        ),
    )
    def pool(hs_hbm, nidx_hbm, out_hbm, nidx_v, rows_v, dest_v):
        c = lax.axis_index("c")
        s = lax.axis_index("s")
        wid = s * 2 + c
        pltpu.sync_copy(nidx_hbm.at[wid], nidx_v)
        iota = lax.iota(jnp.int32, 16)

        def body(w, carry):
            pltpu.sync_copy(hs_hbm.at[wid, w], rows_v)
            for l in range(WP // 16):
                nv = nidx_v[w, pl.ds(l * 16, 16)]
                spread = k + ((w * WP + l * 16 + iota) & 31)
                dest_v[0, pl.ds(l * 16, 16)] = jnp.where(nv >= 0, nv, spread)
            pltpu.sync_copy(rows_v, out_hbm.at[dest_v.at[0]])
            return carry

        lax.fori_loop(0, nwp, body, 0)

    return pool


def _make_remap():
    RNW = EP // NW // WE

    @functools.partial(
        pl.kernel,
        out_type=(jax.ShapeDtypeStruct((NW, RNW, WE), jnp.int32),
                  jax.ShapeDtypeStruct((NW, RNW, WE), jnp.int32)),
        mesh=_MESH,
        compiler_params=pltpu.CompilerParams(needs_layout_passes=False),
        scratch_types=(
            pltpu.VMEM((NP0 // 128, 128), jnp.int32),
            pltpu.VMEM((RNW, WE), jnp.int32),
            pltpu.VMEM((RNW, WE), jnp.int32),
            pltpu.VMEM((RNW, WE), jnp.int32),
            pltpu.VMEM((RNW, WE), jnp.int32),
        ),
    )
    def remap(src_hbm, dst_hbm, nidx_hbm, srcn_hbm, dstn_hbm,
              nidx_v, src_v, dst_v, srco_v, dsto_v):
        c = lax.axis_index("c")
        s = lax.axis_index("s")
        wid = s * 2 + c
        pltpu.sync_copy(nidx_hbm, nidx_v)
        pltpu.sync_copy(src_hbm.at[wid], src_v)
        pltpu.sync_copy(dst_hbm.at[wid], dst_v)
        iota = lax.iota(jnp.int32, 16)

        def body(w, carry):
            for l in range(WE // 16):
                i16 = src_v[w, pl.ds(l * 16, 16)]
                j16 = dst_v[w, pl.ds(l * 16, 16)]
                s1 = plsc.load_gather(nidx_v, [i16 >> 7, i16 & 127])
                d1 = plsc.load_gather(nidx_v, [j16 >> 7, j16 & 127])
                keep = (s1 >= 0) & (d1 >= 0)
                e16 = w * WE + l * 16 + iota
                srco_v[w, pl.ds(l * 16, 16)] = jnp.where(keep, s1, e16 & 1023)
                dsto_v[w, pl.ds(l * 16, 16)] = jnp.where(keep, d1, NP1)
            return carry

        lax.fori_loop(0, RNW, body, 0)
        pltpu.sync_copy(srco_v, srcn_hbm.at[wid])
        pltpu.sync_copy(dsto_v, dstn_hbm.at[wid])

    return remap


_REMAP = None


def _conv_body(part_ref, deg_ref, x_ref, wl_ref, wr_ref, b_ref, *rest, relu, score):
    if score:
        p_ref, h_ref, s_ref, hs_ref = rest
    else:
        (h_ref,) = rest
    agg = part_ref[...]
    dg = jnp.maximum(deg_ref[:, 0], 1.0)
    mean = agg / dg[:, None]
    h = (jnp.dot(mean, wl_ref[...], preferred_element_type=jnp.float32)
         + jnp.dot(x_ref[...], wr_ref[...], preferred_element_type=jnp.float32)
         + b_ref[0][None, :])
    if relu:
        h = jnp.maximum(h, 0.0)
    h_ref[...] = h
    if score:
        pv = p_ref[0]
        nrm = jnp.sqrt(jnp.sum(pv * pv)) + 1e-16
        sc = jnp.tanh(jnp.dot(h, (pv / nrm)[:, None],
                              preferred_element_type=jnp.float32)[:, 0])
        s_ref[0, :] = sc
        hs_ref[...] = h * sc[:, None]


@functools.lru_cache(maxsize=None)
def _make_conv(n_pad: int, relu: bool, score: bool):
    BR = 256
    half = n_pad // 2
    n_acc = half + TR
    nb = half // BR
    nba = n_acc // BR
    grid = (n_pad // BR,)
    in_specs = [
        pl.BlockSpec((BR, 128), lambda i: ((i // nb) * nba + i % nb, 0)),
        pl.BlockSpec((BR, 128), lambda i: ((i // nb) * nba + i % nb, 0)),
        pl.BlockSpec((BR, 128), lambda i: (i, 0)),
        pl.BlockSpec((128, 128), lambda i: (0, 0)),
        pl.BlockSpec((128, 128), lambda i: (0, 0)),
        pl.BlockSpec((1, 128), lambda i: (0, 0)),
    ]
    out_shape = [jax.ShapeDtypeStruct((n_pad, 128), jnp.float32)]
    out_specs = [pl.BlockSpec((BR, 128), lambda i: (i, 0))]
    if score:
        in_specs.append(pl.BlockSpec((1, 128), lambda i: (0, 0)))
        out_shape += [jax.ShapeDtypeStruct((1, n_pad), jnp.float32),
                      jax.ShapeDtypeStruct((n_pad, 128), jnp.float32)]
        out_specs += [pl.BlockSpec((1, BR), lambda i: (0, i)),
                      pl.BlockSpec((BR, 128), lambda i: (i, 0))]
    return pl.pallas_call(
        functools.partial(_conv_body, relu=relu, score=score),
        grid=grid, in_specs=in_specs,
        out_shape=out_shape if score else out_shape[0],
        out_specs=out_specs if score else out_specs[0],
    )


@functools.lru_cache(maxsize=None)
def _make_rank(n_pad: int, n_real: int, k: int):
    BV = 512
    CJ = 512

    def body(srow_ref, scol_ref, out_ref):
        i = pl.program_id(0)
        sv = srow_ref[0:1, :]
        vi = jax.lax.broadcasted_iota(jnp.int32, (1, BV), 1) + i * BV
        sv = jnp.where(vi < n_real, sv, -3.0)

        def inner(jc, acc):
            sj = scol_ref[pl.ds(jc * CJ, CJ), 0:1]
            ji = jax.lax.broadcasted_iota(jnp.int32, (CJ, 1), 0) + jc * CJ
            sj = jnp.where(ji < n_real, sj, -3.0)
            gt = (sj > sv).astype(jnp.int32)
            eq = ((sj == sv) & (ji < vi)).astype(jnp.int32)
            return acc + jnp.sum(gt + eq, axis=0, keepdims=True)

        acc = lax.fori_loop(0, n_pad // CJ, inner,
                            jnp.zeros((1, BV), jnp.int32))
        out_ref[...] = jnp.where((acc < k) & (vi < n_real), acc, -1)

    return pl.pallas_call(
        body,
        grid=(n_pad // BV,),
        in_specs=[pl.BlockSpec((1, BV), lambda i: (0, i)),
                  pl.BlockSpec((n_pad, 1), lambda i: (0, 0))],
        out_shape=jax.ShapeDtypeStruct((1, n_pad), jnp.int32),
        out_specs=pl.BlockSpec((1, BV), lambda i: (0, i)),
    )


@functools.lru_cache(maxsize=None)
def _make_cum(m: int, k: int):
    def body(y_ref, w_ref, out_ref):
        yw = jnp.dot(y_ref[...], w_ref[...], preferred_element_type=jnp.float32)
        i2 = jax.lax.broadcasted_iota(jnp.int32, (m, m), 0)
        j2 = jax.lax.broadcasted_iota(jnp.int32, (m, m), 1)
        mm = jnp.where((j2 <= i2) & (i2 < k),
                       1.0 / (i2.astype(jnp.float32) + 1.0), 0.0)
        out_ref[...] = jnp.dot(mm, yw, preferred_element_type=jnp.float32)

    return pl.pallas_call(
        body,
        out_shape=jax.ShapeDtypeStruct((m, 128), jnp.float32),
    )


@functools.lru_cache(maxsize=None)
def _make_up(n_pad: int, relu: bool):
    BR = 640

    def body(a_ref, x_ref, wr_ref, b_ref, out_ref):
        sel = jnp.where(pl.program_id(0) == 0, 1.0, 0.0)
        r = (jnp.dot(x_ref[...], wr_ref[...], preferred_element_type=jnp.float32)
             + b_ref[0][None, :] + sel * a_ref[...])
        if relu:
            r = jnp.maximum(r, 0.0)
        out_ref[...] = r

    return pl.pallas_call(
        body,
        grid=(n_pad // BR,),
        in_specs=[pl.BlockSpec((BR, 128), lambda i: (0, 0)),
                  pl.BlockSpec((BR, 128), lambda i: (i, 0)),
                  pl.BlockSpec((128, 128), lambda i: (0, 0)),
                  pl.BlockSpec((1, 128), lambda i: (0, 0))],
        out_shape=jax.ShapeDtypeStruct((n_pad, 128), jnp.float32),
        out_specs=pl.BlockSpec((BR, 128), lambda i: (i, 0)),
    )


def kernel(x, edge_index, batch, d0_Wl1, d0_Wr1, d0_b1, d0_Wl2, d0_Wr2, d0_b2,
           d1_Wl1, d1_Wr1, d1_b1, d1_Wl2, d1_Wr2, d1_b2, p0, p1,
           u0_Wl1, u0_Wr1, u0_b1, u0_Wl2, u0_Wr2, u0_b2, _stop=None):
    global _REMAP
    xp = jnp.pad(x, ((0, NP0 - N0), (0, 0)))
    pad_src = (jnp.arange(EP - E, dtype=jnp.int32) * 37) % N0
    pad_dst = jnp.full((EP - E,), NP0 - 1, jnp.int32)
    srcp = jnp.concatenate([edge_index[0], pad_src])
    dstp = jnp.concatenate([edge_index[1], pad_dst])
    src16 = srcp.reshape(16, NWIN, WE)
    dst16 = dstp.reshape(16, NWIN, WE)
    src32 = srcp.reshape(NW, EP // NW // WE, WE)
    dst32 = dstp.reshape(NW, EP // NW // WE, WE)
    na0 = NP0 // 2 + TR
    na1 = NP1 // 2 + TR
    z2_0 = jnp.zeros((16, na0 // 16, 128), jnp.float32)
    z2_1 = jnp.zeros((16, na1 // 16, 128), jnp.float32)
    onesb = jnp.ones((WE, 128), jnp.float32)
    b = lambda v: v.reshape(1, 128)

    deg = _make_deg(NP0)(dst16, z2_0, onesb).reshape(2 * na0, 128)
    part = _make_agg(NP0)(xp, src16, dst16, z2_0).reshape(2 * na0, 128)
    if _stop == "agg0":
        return part
    h = _make_conv(NP0, True, False)(part, deg, xp, d0_Wl1, d0_Wr1, b(d0_b1))
    if _stop == "conv01":
        return h
    part2 = _make_agg(NP0)(h, src16, dst16, z2_0).reshape(2 * na0, 128)
    if _stop == "agg02":
        return part2
    h, s_row, hs = _make_conv(NP0, False, True)(
        part2, deg, h, d0_Wl2, d0_Wr2, b(d0_b2), b(p0))
    if _stop == "conv02":
        return hs
    nidx = _make_rank(NP0, N0, K1)(s_row, s_row.reshape(NP0, 1))
    nidx1d = nidx.reshape(NP0)
    if _stop == "rank0":
        return nidx
    x1new = _make_pool(NP0, K1)(hs.reshape(NW, NP0 // NW // WP, WP, 128),
                                nidx1d.reshape(NW, NP0 // NW // WP, WP))
    if _stop == "pool0":
        return x1new
    if _REMAP is None:
        _REMAP = _make_remap()
    nidx1d, x1new = lax.optimization_barrier((nidx1d, x1new))
    srcn, dstn = _REMAP(src32, dst32, nidx1d.reshape(NP0 // 128, 128))
    srcn = srcn.reshape(16, NWIN, WE)
    dstn = dstn.reshape(16, NWIN, WE)
    if _stop == "remap":
        return srcn
    x1p = jnp.pad(x1new[:K1], ((0, NP1 - K1), (0, 0)))

    deg = _make_deg(NP1)(dstn, z2_1, onesb).reshape(2 * na1, 128)
    part = _make_agg(NP1)(x1p, srcn, dstn, z2_1).reshape(2 * na1, 128)
    if _stop == "agg1":
        return part
    h = _make_conv(NP1, True, False)(part, deg, x1p, d1_Wl1, d1_Wr1, b(d1_b1))
    part2 = _make_agg(NP1)(h, srcn, dstn, z2_1).reshape(2 * na1, 128)
    h, s_row, hs = _make_conv(NP1, False, True)(
        part2, deg, h, d1_Wl2, d1_Wr2, b(d1_b2), b(p1))
    nidx2 = _make_rank(NP1, K1, K2)(s_row, s_row.reshape(NP1, 1))
    nidx2, hs = lax.optimization_barrier((nidx2, hs))
    x2new = _make_pool(NP1, K2)(hs.reshape(NW, NP1 // NW // WP, WP, 128),
                                nidx2.reshape(NP1).reshape(NW, NP1 // NW // WP, WP))
    if _stop == "lvl1":
        return x2new

    A = _make_cum(640, K2)(x2new[:640], u0_Wl1)
    hup = _make_up(NP0, True)(A, xp, u0_Wr1, b(u0_b1))
    C = _make_cum(640, K2)(hup[:640], u0_Wl2)
    out = _make_up(NP0, False)(C, hup, u0_Wr2, b(u0_b2))
    return out[:N0]


# speedup vs baseline: 9.7118x; 1.3304x over previous
import functools
import numpy as np
import jax
import jax.numpy as jnp
from jax import lax
from jax.experimental import pallas as pl
from jax.experimental.pallas import tpu as pltpu
from jax.experimental.pallas import tpu_sc as plsc

N0 = 10000
E = 320000
EP = 327680
K1 = 2500
K2 = 625
NP0 = 10240
NP1 = 2560
NW = 32
WE = 128
NWIN = EP // 16 // WE
WP = 80
TR = 256

_MESH = plsc.VectorSubcoreMesh(core_axis_name="c", subcore_axis_name="s")


@functools.lru_cache(maxsize=None)
def _make_agg(n_pad: int):
    half = n_pad // 2
    n_acc = half + TR
    rps = n_acc // 16

    @functools.partial(
        pl.kernel,
        out_type=jax.ShapeDtypeStruct((2, 16, rps, 128), jnp.float32),
        mesh=_MESH,
        scratch_types=(
            pltpu.VMEM((NWIN, WE), jnp.int32),
            pltpu.VMEM((NWIN, WE), jnp.int32),
            pltpu.VMEM((1, WE), jnp.int32),
            pltpu.VMEM((1, WE), jnp.int32),
            pltpu.VMEM((1, WE), jnp.int32),
            pltpu.VMEM((WE, 128), jnp.float32),
            pltpu.VMEM((WE, 128), jnp.float32),
            pltpu.VMEM_SHARED((n_acc, 128), jnp.float32),
            pltpu.SemaphoreType.DMA,
            pltpu.SemaphoreType.DMA,
        ),
    )
    def agg(x_hbm, src_hbm, dst_hbm, z2_hbm, part_hbm,
            src_v, dst_v, dloc_v, srcw0, srcw1, rows0, rows1, acc_sh,
            sem0, sem1):
        c = lax.axis_index("c")
        s = lax.axis_index("s")
        base = c * half
        r0 = s * rps
        pltpu.sync_copy(z2_hbm.at[s], acc_sh.at[pl.ds(r0, rps)])
        pltpu.sync_copy(src_hbm.at[s], src_v)
        pltpu.sync_copy(dst_hbm.at[s], dst_v)
        plsc.subcore_barrier()
        iota = lax.iota(jnp.int32, 16)

        def start(w, srcw, buf, sem):
            for l in range(WE // 16):
                srcw[0, pl.ds(l * 16, 16)] = src_v[w, pl.ds(l * 16, 16)]
            pltpu.async_copy(x_hbm.at[srcw.at[0]], buf, sem)

        def finish(w, srcw, buf, sem):
            pltpu.make_async_copy(x_hbm.at[srcw.at[0]], buf, sem).wait()
            for l in range(WE // 16):
                d16 = dst_v[w, pl.ds(l * 16, 16)] - base
                ok = (d16 >= 0) & (d16 < half)
                trash = half + ((l * 16 + iota) & (TR - 1))
                dloc_v[0, pl.ds(l * 16, 16)] = jnp.where(ok, d16, trash)
            pltpu.sync_copy(buf, acc_sh.at[dloc_v.at[0]], add=True)

        start(0, srcw0, rows0, sem0)

        def body(i, carry):
            w = 2 * i

            @pl.when(w + 1 < NWIN)
            def _():
                start(w + 1, srcw1, rows1, sem1)

            finish(w, srcw0, rows0, sem0)

            @pl.when(w + 2 < NWIN)
            def _():
                start(w + 2, srcw0, rows0, sem0)

            @pl.when(w + 1 < NWIN)
            def _():
                finish(w + 1, srcw1, rows1, sem1)

            return carry

        lax.fori_loop(0, (NWIN + 1) // 2, body, 0)
        plsc.subcore_barrier()
        pltpu.sync_copy(acc_sh.at[pl.ds(r0, rps)], part_hbm.at[c, s])

    return agg


@functools.lru_cache(maxsize=None)
def _make_deg(n_pad: int):
    half = n_pad // 2
    n_acc = half + TR
    rps = n_acc // 16

    @functools.partial(
        pl.kernel,
        out_type=jax.ShapeDtypeStruct((2, 16, rps, 128), jnp.float32),
        mesh=_MESH,
        scratch_types=(
            pltpu.VMEM((NWIN, WE), jnp.int32),
            pltpu.VMEM((1, WE), jnp.int32),
            pltpu.VMEM((WE, 128), jnp.float32),
            pltpu.VMEM_SHARED((n_acc, 128), jnp.float32),
        ),
    )
    def deg(dst_hbm, z2_hbm, ones_hbm, deg_hbm, dst_v, dloc_v, ones_v, acc_sh):
        c = lax.axis_index("c")
        s = lax.axis_index("s")
        base = c * half
        r0 = s * rps
        pltpu.sync_copy(z2_hbm.at[s], acc_sh.at[pl.ds(r0, rps)])
        pltpu.sync_copy(ones_hbm, ones_v)
        pltpu.sync_copy(dst_hbm.at[s], dst_v)
        plsc.subcore_barrier()
        iota = lax.iota(jnp.int32, 16)

        def body(w, carry):
            for l in range(WE // 16):
                d16 = dst_v[w, pl.ds(l * 16, 16)] - base
                ok = (d16 >= 0) & (d16 < half)
                trash = half + ((l * 16 + iota) & (TR - 1))
                dloc_v[0, pl.ds(l * 16, 16)] = jnp.where(ok, d16, trash)
            pltpu.sync_copy(ones_v, acc_sh.at[dloc_v.at[0]], add=True)
            return carry

        lax.fori_loop(0, NWIN, body, 0)
        plsc.subcore_barrier()
        pltpu.sync_copy(acc_sh.at[pl.ds(r0, rps)], deg_hbm.at[c, s])

    return deg


@functools.lru_cache(maxsize=None)
def _make_pool(n_pad: int, k: int):
    nwp = n_pad // NW // WP

    @functools.partial(
        pl.kernel,
        out_type=jax.ShapeDtypeStruct((k + 32, 128), jnp.float32),
        mesh=_MESH,
        scratch_types=(
            pltpu.VMEM((nwp, WP), jnp.int32),
            pltpu.VMEM((WP, 128), jnp.float32),
            pltpu.VMEM((1, WP), jnp.int32),
        ),
    )
    def pool(hs_hbm, nidx_hbm, out_hbm, nidx_v, rows_v, dest_v):
        c = lax.axis_index("c")
        s = lax.axis_index("s")
        wid = s * 2 + c
        pltpu.sync_copy(nidx_hbm.at[wid], nidx_v)
        iota = lax.iota(jnp.int32, 16)

        def body(w, carry):
            pltpu.sync_copy(hs_hbm.at[wid, w], rows_v)
            for l in range(WP // 16):
                nv = nidx_v[w, pl.ds(l * 16, 16)]
                spread = k + ((w * WP + l * 16 + iota) & 31)
                dest_v[0, pl.ds(l * 16, 16)] = jnp.where(nv >= 0, nv, spread)
            pltpu.sync_copy(rows_v, out_hbm.at[dest_v.at[0]])
            return carry

        lax.fori_loop(0, nwp, body, 0)

    return pool


def _make_remap():
    RNW = EP // NW // WE

    @functools.partial(
        pl.kernel,
        out_type=(jax.ShapeDtypeStruct((NW, RNW, WE), jnp.int32),
                  jax.ShapeDtypeStruct((NW, RNW, WE), jnp.int32)),
        mesh=_MESH,
        compiler_params=pltpu.CompilerParams(needs_layout_passes=False),
        scratch_types=(
            pltpu.VMEM((NP0 // 128, 128), jnp.int32),
            pltpu.VMEM((RNW, WE), jnp.int32),
            pltpu.VMEM((RNW, WE), jnp.int32),
            pltpu.VMEM((RNW, WE), jnp.int32),
            pltpu.VMEM((RNW, WE), jnp.int32),
        ),
    )
    def remap(src_hbm, dst_hbm, nidx_hbm, srcn_hbm, dstn_hbm,
              nidx_v, src_v, dst_v, srco_v, dsto_v):
        c = lax.axis_index("c")
        s = lax.axis_index("s")
        wid = s * 2 + c
        pltpu.sync_copy(nidx_hbm, nidx_v)
        pltpu.sync_copy(src_hbm.at[wid], src_v)
        pltpu.sync_copy(dst_hbm.at[wid], dst_v)
        iota = lax.iota(jnp.int32, 16)

        def body(w, carry):
            for l in range(WE // 16):
                i16 = src_v[w, pl.ds(l * 16, 16)]
                j16 = dst_v[w, pl.ds(l * 16, 16)]
                s1 = plsc.load_gather(nidx_v, [i16 >> 7, i16 & 127])
                d1 = plsc.load_gather(nidx_v, [j16 >> 7, j16 & 127])
                keep = (s1 >= 0) & (d1 >= 0)
                e16 = w * WE + l * 16 + iota
                srco_v[w, pl.ds(l * 16, 16)] = jnp.where(keep, s1, e16 & 1023)
                dsto_v[w, pl.ds(l * 16, 16)] = jnp.where(keep, d1, NP1)
            return carry

        lax.fori_loop(0, RNW, body, 0)
        pltpu.sync_copy(srco_v, srcn_hbm.at[wid])
        pltpu.sync_copy(dsto_v, dstn_hbm.at[wid])

    return remap


_REMAP = None


def _conv_body(part_ref, deg_ref, x_ref, wl_ref, wr_ref, b_ref, *rest, relu, score):
    if score:
        p_ref, h_ref, s_ref, hs_ref = rest
    else:
        (h_ref,) = rest
    agg = part_ref[...]
    dg = jnp.maximum(deg_ref[:, 0], 1.0)
    mean = agg / dg[:, None]
    h = (jnp.dot(mean, wl_ref[...], preferred_element_type=jnp.float32)
         + jnp.dot(x_ref[...], wr_ref[...], preferred_element_type=jnp.float32)
         + b_ref[0][None, :])
    if relu:
        h = jnp.maximum(h, 0.0)
    h_ref[...] = h
    if score:
        pv = p_ref[0]
        nrm = jnp.sqrt(jnp.sum(pv * pv)) + 1e-16
        sc = jnp.tanh(jnp.dot(h, (pv / nrm)[:, None],
                              preferred_element_type=jnp.float32)[:, 0])
        s_ref[0, :] = sc
        hs_ref[...] = h * sc[:, None]


@functools.lru_cache(maxsize=None)
def _make_conv(n_pad: int, relu: bool, score: bool):
    BR = 256
    half = n_pad // 2
    n_acc = half + TR
    nb = half // BR
    nba = n_acc // BR
    grid = (n_pad // BR,)
    in_specs = [
        pl.BlockSpec((BR, 128), lambda i: ((i // nb) * nba + i % nb, 0)),
        pl.BlockSpec((BR, 128), lambda i: ((i // nb) * nba + i % nb, 0)),
        pl.BlockSpec((BR, 128), lambda i: (i, 0)),
        pl.BlockSpec((128, 128), lambda i: (0, 0)),
        pl.BlockSpec((128, 128), lambda i: (0, 0)),
        pl.BlockSpec((1, 128), lambda i: (0, 0)),
    ]
    out_shape = [jax.ShapeDtypeStruct((n_pad, 128), jnp.float32)]
    out_specs = [pl.BlockSpec((BR, 128), lambda i: (i, 0))]
    if score:
        in_specs.append(pl.BlockSpec((1, 128), lambda i: (0, 0)))
        out_shape += [jax.ShapeDtypeStruct((1, n_pad), jnp.float32),
                      jax.ShapeDtypeStruct((n_pad, 128), jnp.float32)]
        out_specs += [pl.BlockSpec((1, BR), lambda i: (0, i)),
                      pl.BlockSpec((BR, 128), lambda i: (i, 0))]
    return pl.pallas_call(
        functools.partial(_conv_body, relu=relu, score=score),
        grid=grid, in_specs=in_specs,
        out_shape=out_shape if score else out_shape[0],
        out_specs=out_specs if score else out_specs[0],
    )


@functools.lru_cache(maxsize=None)
def _make_rank(n_pad: int, n_real: int, k: int):
    BV = 512
    CJ = 512

    def body(srow_ref, scol_ref, out_ref):
        i = pl.program_id(0)
        sv = srow_ref[0:1, :]
        vi = jax.lax.broadcasted_iota(jnp.int32, (1, BV), 1) + i * BV
        sv = jnp.where(vi < n_real, sv, -3.0)

        def inner(jc, acc):
            sj = scol_ref[pl.ds(jc * CJ, CJ), 0:1]
            ji = jax.lax.broadcasted_iota(jnp.int32, (CJ, 1), 0) + jc * CJ
            sj = jnp.where(ji < n_real, sj, -3.0)
            gt = (sj > sv).astype(jnp.int32)
            eq = ((sj == sv) & (ji < vi)).astype(jnp.int32)
            return acc + jnp.sum(gt + eq, axis=0, keepdims=True)

        acc = lax.fori_loop(0, n_pad // CJ, inner,
                            jnp.zeros((1, BV), jnp.int32))
        out_ref[...] = jnp.where((acc < k) & (vi < n_real), acc, -1)

    return pl.pallas_call(
        body,
        grid=(n_pad // BV,),
        in_specs=[pl.BlockSpec((1, BV), lambda i: (0, i)),
                  pl.BlockSpec((n_pad, 1), lambda i: (0, 0))],
        out_shape=jax.ShapeDtypeStruct((1, n_pad), jnp.int32),
        out_specs=pl.BlockSpec((1, BV), lambda i: (0, i)),
    )


@functools.lru_cache(maxsize=None)
def _make_cum(m: int, k: int):
    def body(y_ref, w_ref, out_ref):
        yw = jnp.dot(y_ref[...], w_ref[...], preferred_element_type=jnp.float32)
        i2 = jax.lax.broadcasted_iota(jnp.int32, (m, m), 0)
        j2 = jax.lax.broadcasted_iota(jnp.int32, (m, m), 1)
        mm = jnp.where((j2 <= i2) & (i2 < k),
                       1.0 / (i2.astype(jnp.float32) + 1.0), 0.0)
        out_ref[...] = jnp.dot(mm, yw, preferred_element_type=jnp.float32)

    return pl.pallas_call(
        body,
        out_shape=jax.ShapeDtypeStruct((m, 128), jnp.float32),
    )


@functools.lru_cache(maxsize=None)
def _make_up(n_pad: int, relu: bool):
    BR = 640

    def body(a_ref, x_ref, wr_ref, b_ref, out_ref):
        sel = jnp.where(pl.program_id(0) == 0, 1.0, 0.0)
        r = (jnp.dot(x_ref[...], wr_ref[...], preferred_element_type=jnp.float32)
             + b_ref[0][None, :] + sel * a_ref[...])
        if relu:
            r = jnp.maximum(r, 0.0)
        out_ref[...] = r

    return pl.pallas_call(
        body,
        grid=(n_pad // BR,),
        in_specs=[pl.BlockSpec((BR, 128), lambda i: (0, 0)),
                  pl.BlockSpec((BR, 128), lambda i: (i, 0)),
                  pl.BlockSpec((128, 128), lambda i: (0, 0)),
                  pl.BlockSpec((1, 128), lambda i: (0, 0))],
        out_shape=jax.ShapeDtypeStruct((n_pad, 128), jnp.float32),
        out_specs=pl.BlockSpec((BR, 128), lambda i: (i, 0)),
    )


def kernel(x, edge_index, batch, d0_Wl1, d0_Wr1, d0_b1, d0_Wl2, d0_Wr2, d0_b2,
           d1_Wl1, d1_Wr1, d1_b1, d1_Wl2, d1_Wr2, d1_b2, p0, p1,
           u0_Wl1, u0_Wr1, u0_b1, u0_Wl2, u0_Wr2, u0_b2, _stop=None):
    global _REMAP
    xp = jnp.pad(x, ((0, NP0 - N0), (0, 0)))
    pad_src = (jnp.arange(EP - E, dtype=jnp.int32) * 37) % N0
    pad_dst = jnp.full((EP - E,), NP0 - 1, jnp.int32)
    srcp = jnp.concatenate([edge_index[0], pad_src])
    dstp = jnp.concatenate([edge_index[1], pad_dst])
    src16 = srcp.reshape(16, NWIN, WE)
    dst16 = dstp.reshape(16, NWIN, WE)
    src32 = srcp.reshape(NW, EP // NW // WE, WE)
    dst32 = dstp.reshape(NW, EP // NW // WE, WE)
    na0 = NP0 // 2 + TR
    na1 = NP1 // 2 + TR
    z2_0 = jnp.zeros((16, na0 // 16, 128), jnp.float32)
    z2_1 = jnp.zeros((16, na1 // 16, 128), jnp.float32)
    onesb = jnp.ones((WE, 128), jnp.float32)
    b = lambda v: v.reshape(1, 128)

    deg = _make_deg(NP0)(dst16, z2_0, onesb).reshape(2 * na0, 128)
    part = _make_agg(NP0)(xp, src16, dst16, z2_0).reshape(2 * na0, 128)
    if _stop == "agg0":
        return part
    h = _make_conv(NP0, True, False)(part, deg, xp, d0_Wl1, d0_Wr1, b(d0_b1))
    if _stop == "conv01":
        return h
    part2 = _make_agg(NP0)(h, src16, dst16, z2_0).reshape(2 * na0, 128)
    if _stop == "agg02":
        return part2
    h, s_row, hs = _make_conv(NP0, False, True)(
        part2, deg, h, d0_Wl2, d0_Wr2, b(d0_b2), b(p0))
    if _stop == "conv02":
        return hs
    nidx = _make_rank(NP0, N0, K1)(s_row, s_row.reshape(NP0, 1))
    nidx1d = nidx.reshape(NP0)
    if _stop == "rank0":
        return nidx
    x1new = _make_pool(NP0, K1)(hs.reshape(NW, NP0 // NW // WP, WP, 128),
                                nidx1d.reshape(NW, NP0 // NW // WP, WP))
    if _stop == "pool0":
        return x1new
    if _REMAP is None:
        _REMAP = _make_remap()
    nidx1d, x1new = lax.optimization_barrier((nidx1d, x1new))
    srcn, dstn = _REMAP(src32, dst32, nidx1d.reshape(NP0 // 128, 128))
    srcn = srcn.reshape(16, NWIN, WE)
    dstn = dstn.reshape(16, NWIN, WE)
    if _stop == "remap":
        return srcn
    x1p = jnp.pad(x1new[:K1], ((0, NP1 - K1), (0, 0)))

    deg = _make_deg(NP1)(dstn, z2_1, onesb).reshape(2 * na1, 128)
    part = _make_agg(NP1)(x1p, srcn, dstn, z2_1).reshape(2 * na1, 128)
    if _stop == "agg1":
        return part
    h = _make_conv(NP1, True, False)(part, deg, x1p, d1_Wl1, d1_Wr1, b(d1_b1))
    part2 = _make_agg(NP1)(h, srcn, dstn, z2_1).reshape(2 * na1, 128)
    h, s_row, hs = _make_conv(NP1, False, True)(
        part2, deg, h, d1_Wl2, d1_Wr2, b(d1_b2), b(p1))
    nidx2 = _make_rank(NP1, K1, K2)(s_row, s_row.reshape(NP1, 1))
    nidx2, hs = lax.optimization_barrier((nidx2, hs))
    x2new = _make_pool(NP1, K2)(hs.reshape(NW, NP1 // NW // WP, WP, 128),
                                nidx2.reshape(NP1).reshape(NW, NP1 // NW // WP, WP))
    if _stop == "lvl1":
        return x2new

    A = _make_cum(640, K2)(x2new[:640], u0_Wl1)
    hup = _make_up(NP0, True)(A, xp, u0_Wr1, b(u0_b1))
    C = _make_cum(640, K2)(hup[:640], u0_Wl2)
    out = _make_up(NP0, False)(C, hup, u0_Wr2, b(u0_b2))
    return out[:N0]


# speedup vs baseline: 10.2236x; 1.0527x over previous
import functools
import numpy as np
import jax
import jax.numpy as jnp
from jax import lax
from jax.experimental import pallas as pl
from jax.experimental.pallas import tpu as pltpu
from jax.experimental.pallas import tpu_sc as plsc

N0 = 10000
E = 320000
EP = 327680
K1 = 2500
K2 = 625
NP0 = 10240
NP1 = 2560
NW = 32
WE = 128
NWIN = EP // 16 // WE
WP = 80
TR = 512

_MESH = plsc.VectorSubcoreMesh(core_axis_name="c", subcore_axis_name="s")


@functools.lru_cache(maxsize=None)
def _make_agg(n_pad: int):
    half = n_pad // 2
    n_acc = half + TR
    rps = n_acc // 16

    @functools.partial(
        pl.kernel,
        out_type=jax.ShapeDtypeStruct((2, 16, rps, 128), jnp.float32),
        mesh=_MESH,
        scratch_types=(
            pltpu.VMEM((NWIN, WE), jnp.int32),
            pltpu.VMEM((NWIN, WE), jnp.int32),
            pltpu.VMEM((1, WE), jnp.int32),
            pltpu.VMEM((1, WE), jnp.int32),
            pltpu.VMEM((1, WE), jnp.int32),
            pltpu.VMEM((WE, 128), jnp.float32),
            pltpu.VMEM((WE, 128), jnp.float32),
            pltpu.VMEM_SHARED((n_acc, 128), jnp.float32),
            pltpu.SemaphoreType.DMA,
            pltpu.SemaphoreType.DMA,
        ),
    )
    def agg(x_hbm, src_hbm, dst_hbm, z2_hbm, part_hbm,
            src_v, dst_v, dloc_v, srcw0, srcw1, rows0, rows1, acc_sh,
            sem0, sem1):
        c = lax.axis_index("c")
        s = lax.axis_index("s")
        base = c * half
        r0 = s * rps
        pltpu.sync_copy(z2_hbm.at[s], acc_sh.at[pl.ds(r0, rps)])
        pltpu.sync_copy(src_hbm.at[s], src_v)
        pltpu.sync_copy(dst_hbm.at[s], dst_v)
        plsc.subcore_barrier()
        iota = lax.iota(jnp.int32, 16)

        def start(w, srcw, buf, sem):
            for l in range(WE // 16):
                srcw[0, pl.ds(l * 16, 16)] = src_v[w, pl.ds(l * 16, 16)]
            pltpu.async_copy(x_hbm.at[srcw.at[0]], buf, sem)

        def finish(w, srcw, buf, sem):
            pltpu.make_async_copy(x_hbm.at[srcw.at[0]], buf, sem).wait()
            for l in range(WE // 16):
                d16 = dst_v[w, pl.ds(l * 16, 16)] - base
                ok = (d16 >= 0) & (d16 < half)
                trash = half + ((w * WE + l * 16 + iota) & (TR - 1))
                dloc_v[0, pl.ds(l * 16, 16)] = jnp.where(ok, d16, trash)
            pltpu.sync_copy(buf, acc_sh.at[dloc_v.at[0]], add=True)

        start(0, srcw0, rows0, sem0)

        def body(i, carry):
            w = 2 * i

            @pl.when(w + 1 < NWIN)
            def _():
                start(w + 1, srcw1, rows1, sem1)

            finish(w, srcw0, rows0, sem0)

            @pl.when(w + 2 < NWIN)
            def _():
                start(w + 2, srcw0, rows0, sem0)

            @pl.when(w + 1 < NWIN)
            def _():
                finish(w + 1, srcw1, rows1, sem1)

            return carry

        lax.fori_loop(0, (NWIN + 1) // 2, body, 0)
        plsc.subcore_barrier()
        pltpu.sync_copy(acc_sh.at[pl.ds(r0, rps)], part_hbm.at[c, s])

    return agg


@functools.lru_cache(maxsize=None)
def _make_deg(n_pad: int):
    half = n_pad // 2
    n_acc = half + TR
    rps = n_acc // 16

    @functools.partial(
        pl.kernel,
        out_type=jax.ShapeDtypeStruct((2, 16, rps, 128), jnp.float32),
        mesh=_MESH,
        scratch_types=(
            pltpu.VMEM((NWIN, WE), jnp.int32),
            pltpu.VMEM((1, WE), jnp.int32),
            pltpu.VMEM((WE, 128), jnp.float32),
            pltpu.VMEM_SHARED((n_acc, 128), jnp.float32),
        ),
    )
    def deg(dst_hbm, z2_hbm, ones_hbm, deg_hbm, dst_v, dloc_v, ones_v, acc_sh):
        c = lax.axis_index("c")
        s = lax.axis_index("s")
        base = c * half
        r0 = s * rps
        pltpu.sync_copy(z2_hbm.at[s], acc_sh.at[pl.ds(r0, rps)])
        pltpu.sync_copy(ones_hbm, ones_v)
        pltpu.sync_copy(dst_hbm.at[s], dst_v)
        plsc.subcore_barrier()
        iota = lax.iota(jnp.int32, 16)

        def body(w, carry):
            for l in range(WE // 16):
                d16 = dst_v[w, pl.ds(l * 16, 16)] - base
                ok = (d16 >= 0) & (d16 < half)
                trash = half + ((w * WE + l * 16 + iota) & (TR - 1))
                dloc_v[0, pl.ds(l * 16, 16)] = jnp.where(ok, d16, trash)
            pltpu.sync_copy(ones_v, acc_sh.at[dloc_v.at[0]], add=True)
            return carry

        lax.fori_loop(0, NWIN, body, 0)
        plsc.subcore_barrier()
        pltpu.sync_copy(acc_sh.at[pl.ds(r0, rps)], deg_hbm.at[c, s])

    return deg


@functools.lru_cache(maxsize=None)
def _make_pool(n_pad: int, k: int):
    nwp = n_pad // NW // WP

    @functools.partial(
        pl.kernel,
        out_type=jax.ShapeDtypeStruct((k + 32, 128), jnp.float32),
        mesh=_MESH,
        scratch_types=(
            pltpu.VMEM((nwp, WP), jnp.int32),
            pltpu.VMEM((WP, 128), jnp.float32),
            pltpu.VMEM((1, WP), jnp.int32),
        ),
    )
    def pool(hs_hbm, nidx_hbm, out_hbm, nidx_v, rows_v, dest_v):
        c = lax.axis_index("c")
        s = lax.axis_index("s")
        wid = s * 2 + c
        pltpu.sync_copy(nidx_hbm.at[wid], nidx_v)
        iota = lax.iota(jnp.int32, 16)

        def body(w, carry):
            pltpu.sync_copy(hs_hbm.at[wid, w], rows_v)
            for l in range(WP // 16):
                nv = nidx_v[w, pl.ds(l * 16, 16)]
                spread = k + ((w * WP + l * 16 + iota) & 31)
                dest_v[0, pl.ds(l * 16, 16)] = jnp.where(nv >= 0, nv, spread)
            pltpu.sync_copy(rows_v, out_hbm.at[dest_v.at[0]])
            return carry

        lax.fori_loop(0, nwp, body, 0)

    return pool


def _make_remap():
    RNW = EP // NW // WE

    @functools.partial(
        pl.kernel,
        out_type=(jax.ShapeDtypeStruct((NW, RNW, WE), jnp.int32),
                  jax.ShapeDtypeStruct((NW, RNW, WE), jnp.int32)),
        mesh=_MESH,
        compiler_params=pltpu.CompilerParams(needs_layout_passes=False),
        scratch_types=(
            pltpu.VMEM((NP0 // 128, 128), jnp.int32),
            pltpu.VMEM((RNW, WE), jnp.int32),
            pltpu.VMEM((RNW, WE), jnp.int32),
            pltpu.VMEM((RNW, WE), jnp.int32),
            pltpu.VMEM((RNW, WE), jnp.int32),
        ),
    )
    def remap(src_hbm, dst_hbm, nidx_hbm, srcn_hbm, dstn_hbm,
              nidx_v, src_v, dst_v, srco_v, dsto_v):
        c = lax.axis_index("c")
        s = lax.axis_index("s")
        wid = s * 2 + c
        pltpu.sync_copy(nidx_hbm, nidx_v)
        pltpu.sync_copy(src_hbm.at[wid], src_v)
        pltpu.sync_copy(dst_hbm.at[wid], dst_v)
        iota = lax.iota(jnp.int32, 16)

        def body(w, carry):
            for l in range(WE // 16):
                i16 = src_v[w, pl.ds(l * 16, 16)]
                j16 = dst_v[w, pl.ds(l * 16, 16)]
                s1 = plsc.load_gather(nidx_v, [i16 >> 7, i16 & 127])
                d1 = plsc.load_gather(nidx_v, [j16 >> 7, j16 & 127])
                keep = (s1 >= 0) & (d1 >= 0)
                e16 = w * WE + l * 16 + iota
                srco_v[w, pl.ds(l * 16, 16)] = jnp.where(keep, s1, e16 & 2047)
                dsto_v[w, pl.ds(l * 16, 16)] = jnp.where(keep, d1, NP1)
            return carry

        lax.fori_loop(0, RNW, body, 0)
        pltpu.sync_copy(srco_v, srcn_hbm.at[wid])
        pltpu.sync_copy(dsto_v, dstn_hbm.at[wid])

    return remap


_REMAP = None


def _conv_body(part_ref, deg_ref, x_ref, wl_ref, wr_ref, b_ref, *rest, relu, score):
    if score:
        p_ref, h_ref, s_ref, hs_ref = rest
    else:
        (h_ref,) = rest
    agg = part_ref[...]
    dg = jnp.maximum(deg_ref[:, 0], 1.0)
    mean = agg / dg[:, None]
    h = (jnp.dot(mean, wl_ref[...], preferred_element_type=jnp.float32)
         + jnp.dot(x_ref[...], wr_ref[...], preferred_element_type=jnp.float32)
         + b_ref[0][None, :])
    if relu:
        h = jnp.maximum(h, 0.0)
    h_ref[...] = h
    if score:
        pv = p_ref[0]
        nrm = jnp.sqrt(jnp.sum(pv * pv)) + 1e-16
        sc = jnp.tanh(jnp.dot(h, (pv / nrm)[:, None],
                              preferred_element_type=jnp.float32)[:, 0])
        s_ref[0, :] = sc
        hs_ref[...] = h * sc[:, None]


@functools.lru_cache(maxsize=None)
def _make_conv(n_pad: int, relu: bool, score: bool):
    BR = 256
    half = n_pad // 2
    n_acc = half + TR
    nb = half // BR
    nba = n_acc // BR
    grid = (n_pad // BR,)
    in_specs = [
        pl.BlockSpec((BR, 128), lambda i: ((i // nb) * nba + i % nb, 0)),
        pl.BlockSpec((BR, 128), lambda i: ((i // nb) * nba + i % nb, 0)),
        pl.BlockSpec((BR, 128), lambda i: (i, 0)),
        pl.BlockSpec((128, 128), lambda i: (0, 0)),
        pl.BlockSpec((128, 128), lambda i: (0, 0)),
        pl.BlockSpec((1, 128), lambda i: (0, 0)),
    ]
    out_shape = [jax.ShapeDtypeStruct((n_pad, 128), jnp.float32)]
    out_specs = [pl.BlockSpec((BR, 128), lambda i: (i, 0))]
    if score:
        in_specs.append(pl.BlockSpec((1, 128), lambda i: (0, 0)))
        out_shape += [jax.ShapeDtypeStruct((1, n_pad), jnp.float32),
                      jax.ShapeDtypeStruct((n_pad, 128), jnp.float32)]
        out_specs += [pl.BlockSpec((1, BR), lambda i: (0, i)),
                      pl.BlockSpec((BR, 128), lambda i: (i, 0))]
    return pl.pallas_call(
        functools.partial(_conv_body, relu=relu, score=score),
        grid=grid, in_specs=in_specs,
        out_shape=out_shape if score else out_shape[0],
        out_specs=out_specs if score else out_specs[0],
    )


@functools.lru_cache(maxsize=None)
def _make_rank(n_pad: int, n_real: int, k: int):
    BV = 512
    CJ = 512

    def body(srow_ref, scol_ref, out_ref):
        i = pl.program_id(0)
        sv = srow_ref[0:1, :]
        vi = jax.lax.broadcasted_iota(jnp.int32, (1, BV), 1) + i * BV
        sv = jnp.where(vi < n_real, sv, -3.0)

        def inner(jc, acc):
            sj = scol_ref[pl.ds(jc * CJ, CJ), 0:1]
            ji = jax.lax.broadcasted_iota(jnp.int32, (CJ, 1), 0) + jc * CJ
            sj = jnp.where(ji < n_real, sj, -3.0)
            gt = (sj > sv).astype(jnp.int32)
            eq = ((sj == sv) & (ji < vi)).astype(jnp.int32)
            return acc + jnp.sum(gt + eq, axis=0, keepdims=True)

        acc = lax.fori_loop(0, n_pad // CJ, inner,
                            jnp.zeros((1, BV), jnp.int32))
        out_ref[...] = jnp.where((acc < k) & (vi < n_real), acc, -1)

    return pl.pallas_call(
        body,
        grid=(n_pad // BV,),
        in_specs=[pl.BlockSpec((1, BV), lambda i: (0, i)),
                  pl.BlockSpec((n_pad, 1), lambda i: (0, 0))],
        out_shape=jax.ShapeDtypeStruct((1, n_pad), jnp.int32),
        out_specs=pl.BlockSpec((1, BV), lambda i: (0, i)),
    )


@functools.lru_cache(maxsize=None)
def _make_cum(m: int, k: int):
    def body(y_ref, w_ref, out_ref):
        yw = jnp.dot(y_ref[...], w_ref[...], preferred_element_type=jnp.float32)
        i2 = jax.lax.broadcasted_iota(jnp.int32, (m, m), 0)
        j2 = jax.lax.broadcasted_iota(jnp.int32, (m, m), 1)
        mm = jnp.where((j2 <= i2) & (i2 < k),
                       1.0 / (i2.astype(jnp.float32) + 1.0), 0.0)
        out_ref[...] = jnp.dot(mm, yw, preferred_element_type=jnp.float32)

    return pl.pallas_call(
        body,
        out_shape=jax.ShapeDtypeStruct((m, 128), jnp.float32),
    )


@functools.lru_cache(maxsize=None)
def _make_up(n_pad: int, relu: bool):
    BR = 640

    def body(a_ref, x_ref, wr_ref, b_ref, out_ref):
        sel = jnp.where(pl.program_id(0) == 0, 1.0, 0.0)
        r = (jnp.dot(x_ref[...], wr_ref[...], preferred_element_type=jnp.float32)
             + b_ref[0][None, :] + sel * a_ref[...])
        if relu:
            r = jnp.maximum(r, 0.0)
        out_ref[...] = r

    return pl.pallas_call(
        body,
        grid=(n_pad // BR,),
        in_specs=[pl.BlockSpec((BR, 128), lambda i: (0, 0)),
                  pl.BlockSpec((BR, 128), lambda i: (i, 0)),
                  pl.BlockSpec((128, 128), lambda i: (0, 0)),
                  pl.BlockSpec((1, 128), lambda i: (0, 0))],
        out_shape=jax.ShapeDtypeStruct((n_pad, 128), jnp.float32),
        out_specs=pl.BlockSpec((BR, 128), lambda i: (i, 0)),
    )


def kernel(x, edge_index, batch, d0_Wl1, d0_Wr1, d0_b1, d0_Wl2, d0_Wr2, d0_b2,
           d1_Wl1, d1_Wr1, d1_b1, d1_Wl2, d1_Wr2, d1_b2, p0, p1,
           u0_Wl1, u0_Wr1, u0_b1, u0_Wl2, u0_Wr2, u0_b2, _stop=None):
    global _REMAP
    xp = jnp.pad(x, ((0, NP0 - N0), (0, 0)))
    pad_src = (jnp.arange(EP - E, dtype=jnp.int32) * 37) % N0
    pad_dst = jnp.full((EP - E,), NP0 - 1, jnp.int32)
    srcp = jnp.concatenate([edge_index[0], pad_src])
    dstp = jnp.concatenate([edge_index[1], pad_dst])
    src16 = srcp.reshape(16, NWIN, WE)
    dst16 = dstp.reshape(16, NWIN, WE)
    src32 = srcp.reshape(NW, EP // NW // WE, WE)
    dst32 = dstp.reshape(NW, EP // NW // WE, WE)
    na0 = NP0 // 2 + TR
    na1 = NP1 // 2 + TR
    z2_0 = jnp.zeros((16, na0 // 16, 128), jnp.float32)
    z2_1 = jnp.zeros((16, na1 // 16, 128), jnp.float32)
    onesb = jnp.ones((WE, 128), jnp.float32)
    b = lambda v: v.reshape(1, 128)

    deg = _make_deg(NP0)(dst16, z2_0, onesb).reshape(2 * na0, 128)
    part = _make_agg(NP0)(xp, src16, dst16, z2_0).reshape(2 * na0, 128)
    if _stop == "agg0":
        return part
    h = _make_conv(NP0, True, False)(part, deg, xp, d0_Wl1, d0_Wr1, b(d0_b1))
    if _stop == "conv01":
        return h
    part2 = _make_agg(NP0)(h, src16, dst16, z2_0).reshape(2 * na0, 128)
    if _stop == "agg02":
        return part2
    h, s_row, hs = _make_conv(NP0, False, True)(
        part2, deg, h, d0_Wl2, d0_Wr2, b(d0_b2), b(p0))
    if _stop == "conv02":
        return hs
    nidx = _make_rank(NP0, N0, K1)(s_row, s_row.reshape(NP0, 1))
    nidx1d = nidx.reshape(NP0)
    if _stop == "rank0":
        return nidx
    x1new = _make_pool(NP0, K1)(hs.reshape(NW, NP0 // NW // WP, WP, 128),
                                nidx1d.reshape(NW, NP0 // NW // WP, WP))
    if _stop == "pool0":
        return x1new
    if _REMAP is None:
        _REMAP = _make_remap()
    nidx1d, x1new = lax.optimization_barrier((nidx1d, x1new))
    srcn, dstn = _REMAP(src32, dst32, nidx1d.reshape(NP0 // 128, 128))
    srcn = srcn.reshape(16, NWIN, WE)
    dstn = dstn.reshape(16, NWIN, WE)
    if _stop == "remap":
        return srcn
    x1p = jnp.pad(x1new[:K1], ((0, NP1 - K1), (0, 0)))

    deg = _make_deg(NP1)(dstn, z2_1, onesb).reshape(2 * na1, 128)
    part = _make_agg(NP1)(x1p, srcn, dstn, z2_1).reshape(2 * na1, 128)
    if _stop == "agg1":
        return part
    h = _make_conv(NP1, True, False)(part, deg, x1p, d1_Wl1, d1_Wr1, b(d1_b1))
    part2 = _make_agg(NP1)(h, srcn, dstn, z2_1).reshape(2 * na1, 128)
    h, s_row, hs = _make_conv(NP1, False, True)(
        part2, deg, h, d1_Wl2, d1_Wr2, b(d1_b2), b(p1))
    nidx2 = _make_rank(NP1, K1, K2)(s_row, s_row.reshape(NP1, 1))
    nidx2, hs = lax.optimization_barrier((nidx2, hs))
    x2new = _make_pool(NP1, K2)(hs.reshape(NW, NP1 // NW // WP, WP, 128),
                                nidx2.reshape(NP1).reshape(NW, NP1 // NW // WP, WP))
    if _stop == "lvl1":
        return x2new

    A = _make_cum(640, K2)(x2new[:640], u0_Wl1)
    hup = _make_up(NP0, True)(A, xp, u0_Wr1, b(u0_b1))
    C = _make_cum(640, K2)(hup[:640], u0_Wl2)
    out = _make_up(NP0, False)(C, hup, u0_Wr2, b(u0_b2))
    return out[:N0]
